# baseline (speedup 1.0000x reference)
import functools

import jax
import jax.numpy as jnp
from jax import lax
from jax.experimental import pallas as pl
from jax.experimental.pallas import tpu as pltpu
from jax.experimental.pallas import tpu_sc as plsc

N = 10000
N_PAD = 10240
RPT = N_PAD // 32
NPT = N_PAD // 16
E = 320000
CHUNK = 128
CPT = 80
E_PAD = 32 * CHUNK * CPT
EPT = CHUNK * CPT
NBUF = 16
NGRP = CPT // NBUF

_MESH = plsc.VectorSubcoreMesh(core_axis_name="c", subcore_axis_name="s")
_SC_PARAMS = pltpu.CompilerParams(needs_layout_passes=False,
                                  use_tc_tiling_on_sc=False)


def _bit_rsqrt(v):
    i = plsc.bitcast(v, jnp.int32)
    i = jnp.int32(0x5F3759DF) - (i >> 1)
    y = plsc.bitcast(i, jnp.float32)
    for _ in range(3):
        y = y * (1.5 - 0.5 * v * y * y)
    return y


@functools.partial(
    pl.kernel,
    out_type=jax.ShapeDtypeStruct((2 * N_PAD, 16), jnp.float32),
    mesh=_MESH,
    compiler_params=_SC_PARAMS,
    scratch_types=[
        pltpu.VMEM((CPT, CHUNK), jnp.int32),
        pltpu.VMEM((CHUNK, 16), jnp.float32),
        pltpu.VMEM((CHUNK, 16), jnp.float32),
        pltpu.VMEM_SHARED((N_PAD, 16), jnp.float32),
    ] + [pltpu.SemaphoreType.DMA] * NBUF,
)
def _deg_kernel(dst_hbm, out_hbm, dst_all, ones_v, zeros_v, acc, *ssems):
    cid = lax.axis_index("c")
    sid = lax.axis_index("s")
    wid = cid * 16 + sid

    def fill(r, carry):
        ones_v[r, :] = jnp.full((16,), 1.0, jnp.float32)
        zeros_v[r, :] = jnp.zeros((16,), jnp.float32)
        return carry

    lax.fori_loop(0, CHUNK, fill, 0)
    pltpu.sync_copy(dst_hbm.at[pl.ds(wid * CPT, CPT), :], dst_all)
    for k in range(NPT // CHUNK):
        pltpu.sync_copy(zeros_v, acc.at[pl.ds(sid * NPT + k * CHUNK, CHUNK), :])
    plsc.subcore_barrier()

    def body(g, carry):
        descs = []
        for b in range(NBUF):
            descs.append(pltpu.async_copy(
                ones_v, acc.at[dst_all.at[g * NBUF + b]], ssems[b], add=True))
        for d in descs:
            d.wait()
        return carry

    lax.fori_loop(0, NGRP, body, 0)
    plsc.subcore_barrier()
    pltpu.sync_copy(acc.at[pl.ds(sid * NPT, NPT), :],
                    out_hbm.at[pl.ds(cid * N_PAD + sid * NPT, NPT), :])


@functools.partial(
    pl.kernel,
    out_type=(jax.ShapeDtypeStruct((N_PAD, 64), jnp.float32),
              jax.ShapeDtypeStruct((N_PAD, 16), jnp.float32)),
    mesh=_MESH,
    compiler_params=_SC_PARAMS,
    scratch_types=[
        pltpu.VMEM((RPT, 16), jnp.float32),
        pltpu.VMEM((RPT, 16), jnp.float32),
        pltpu.VMEM((RPT, 64), jnp.float32),
    ],
)
def _scale_t1_kernel(deg_hbm, t_hbm, hs_hbm, d_hbm, dg0, dg1, tbuf):
    cid = lax.axis_index("c")
    sid = lax.axis_index("s")
    r0 = (cid * 16 + sid) * RPT
    pltpu.sync_copy(deg_hbm.at[pl.ds(r0, RPT), :], dg0)
    pltpu.sync_copy(deg_hbm.at[pl.ds(N_PAD + r0, RPT), :], dg1)
    pltpu.sync_copy(t_hbm.at[pl.ds(r0, RPT), :], tbuf)

    def row(r, carry):
        deg = dg0[r, :] + dg1[r, :] + 1.0
        dv = _bit_rsqrt(deg)
        dg0[r, :] = dv
        for c in range(64 // 16):
            tbuf[r, pl.ds(c * 16, 16)] = tbuf[r, pl.ds(c * 16, 16)] * dv
        return carry

    lax.fori_loop(0, RPT, row, 0)
    pltpu.sync_copy(tbuf, hs_hbm.at[pl.ds(r0, RPT), :])
    pltpu.sync_copy(dg0, d_hbm.at[pl.ds(r0, RPT), :])


@functools.partial(
    pl.kernel,
    out_type=jax.ShapeDtypeStruct((N_PAD, 32), jnp.float32),
    mesh=_MESH,
    compiler_params=_SC_PARAMS,
    scratch_types=[
        pltpu.VMEM((RPT, 16), jnp.float32),
        pltpu.VMEM((RPT, 32), jnp.float32),
    ],
)
def _scale_t2_kernel(d_hbm, t_hbm, hs_hbm, db, tbuf):
    cid = lax.axis_index("c")
    sid = lax.axis_index("s")
    r0 = (cid * 16 + sid) * RPT
    pltpu.sync_copy(d_hbm.at[pl.ds(r0, RPT), :], db)
    pltpu.sync_copy(t_hbm.at[pl.ds(r0, RPT), :], tbuf)

    def row(r, carry):
        dv = db[r, :]
        for c in range(32 // 16):
            tbuf[r, pl.ds(c * 16, 16)] = tbuf[r, pl.ds(c * 16, 16)] * dv
        return carry

    lax.fori_loop(0, RPT, row, 0)
    pltpu.sync_copy(tbuf, hs_hbm.at[pl.ds(r0, RPT), :])


def _make_agg(D):
    nbuf = 2 if D == 64 else 8
    @functools.partial(
        pl.kernel,
        out_type=jax.ShapeDtypeStruct((2 * N_PAD, D), jnp.float32),
        mesh=_MESH,
        compiler_params=_SC_PARAMS,
        scratch_types=[
            pltpu.VMEM((CPT, CHUNK), jnp.int32),
            pltpu.VMEM((CPT, CHUNK), jnp.int32),
            pltpu.VMEM((nbuf, CHUNK, D), jnp.float32),
            pltpu.VMEM_SHARED((N_PAD, D), jnp.float32),
            pltpu.VMEM_SHARED((N_PAD, D), jnp.float32),
        ] + [pltpu.SemaphoreType.DMA] * (2 * nbuf),
    )
    def _agg(hs_hbm, src_hbm, dst_hbm, out_hbm, src_all, dst_all, rows,
             hs_cache, acc, *sems):
        gsems, ssems = sems[:nbuf], sems[nbuf:]
        cid = lax.axis_index("c")
        sid = lax.axis_index("s")
        wid = cid * 16 + sid

        def fill(r, carry):
            for c in range(D // 16):
                rows[0, r, pl.ds(c * 16, 16)] = jnp.zeros((16,), jnp.float32)
            return carry

        lax.fori_loop(0, CHUNK, fill, 0)
        pltpu.sync_copy(src_hbm.at[pl.ds(wid * CPT, CPT), :], src_all)
        pltpu.sync_copy(dst_hbm.at[pl.ds(wid * CPT, CPT), :], dst_all)
        pltpu.sync_copy(hs_hbm.at[pl.ds(sid * NPT, NPT), :],
                        hs_cache.at[pl.ds(sid * NPT, NPT), :])
        for k in range(NPT // CHUNK):
            pltpu.sync_copy(rows.at[0],
                            acc.at[pl.ds(sid * NPT + k * CHUNK, CHUNK), :])
        plsc.subcore_barrier()
        for b in range(nbuf):
            pltpu.async_copy(hs_cache.at[src_all.at[b]], rows.at[b], gsems[b])

        def body(g, carry):
            descs = []
            for b in range(nbuf):
                j = g * nbuf + b
                pltpu.make_async_copy(hs_cache.at[src_all.at[j]], rows.at[b],
                                      gsems[b]).wait()
                descs.append(pltpu.async_copy(
                    rows.at[b], acc.at[dst_all.at[j]], ssems[b], add=True))
            for d in descs:
                d.wait()

            @pl.when(g + 1 < CPT // nbuf)
            def _refill():
                for b in range(nbuf):
                    j2 = (g + 1) * nbuf + b
                    pltpu.async_copy(hs_cache.at[src_all.at[j2]], rows.at[b],
                                     gsems[b])

            return carry

        lax.fori_loop(0, CPT // nbuf, body, 0)
        plsc.subcore_barrier()
        pltpu.sync_copy(acc.at[pl.ds(sid * NPT, NPT), :],
                        out_hbm.at[pl.ds(cid * N_PAD + sid * NPT, NPT), :])

    return _agg


_agg64 = _make_agg(64)
_agg32 = _make_agg(32)


def _make_combine(D, extra_scale):
    @functools.partial(
        pl.kernel,
        out_type=jax.ShapeDtypeStruct((N_PAD, D), jnp.float32),
        mesh=_MESH,
        compiler_params=_SC_PARAMS,
        scratch_types=[
            pltpu.VMEM((RPT, D), jnp.float32),
            pltpu.VMEM((RPT, D), jnp.float32),
            pltpu.VMEM((RPT, D), jnp.float32),
            pltpu.VMEM((RPT, 16), jnp.float32),
            pltpu.VMEM((D,), jnp.float32),
        ],
    )
    def _comb(agg_hbm, hs_hbm, d_hbm, b_hbm, out_hbm, a0, a1, hsb, db, bb):
        cid = lax.axis_index("c")
        sid = lax.axis_index("s")
        r0 = (cid * 16 + sid) * RPT
        pltpu.sync_copy(agg_hbm.at[pl.ds(r0, RPT), :], a0)
        pltpu.sync_copy(agg_hbm.at[pl.ds(N_PAD + r0, RPT), :], a1)
        pltpu.sync_copy(hs_hbm.at[pl.ds(r0, RPT), :], hsb)
        pltpu.sync_copy(d_hbm.at[pl.ds(r0, RPT), :], db)
        pltpu.sync_copy(b_hbm, bb)

        def row(r, carry):
            dv = db[r, :]
            for c in range(D // 16):
                sl = pl.ds(c * 16, 16)
                s = (a0[r, sl] + a1[r, sl] + hsb[r, sl]) * dv + bb[sl]
                z = jnp.maximum(s, 0.0)
                if extra_scale:
                    z = z * dv
                a0[r, sl] = z
            return carry

        lax.fori_loop(0, RPT, row, 0)
        pltpu.sync_copy(a0, out_hbm.at[pl.ds(r0, RPT), :])

    return _comb


_comb64 = _make_combine(64, False)
_comb32 = _make_combine(32, False)


def _mm_body(x_ref, w_ref, o_ref):
    o_ref[...] = jnp.dot(x_ref[...], w_ref[...],
                         preferred_element_type=jnp.float32)


def _mm(xv, wv):
    return pl.pallas_call(
        _mm_body,
        out_shape=jax.ShapeDtypeStruct((xv.shape[0], wv.shape[1]),
                                       jnp.float32),
    )(xv, wv)


def _mm_bias_body(x_ref, w_ref, b_ref, o_ref):
    o_ref[...] = jnp.dot(x_ref[...], w_ref[...],
                         preferred_element_type=jnp.float32) + b_ref[...]


def _mm_bias(xv, wv, bv):
    return pl.pallas_call(
        _mm_bias_body,
        out_shape=jax.ShapeDtypeStruct((xv.shape[0], wv.shape[1]),
                                       jnp.float32),
    )(xv, wv, bv)


def kernel(x, edge_index, W1, b1, W2, b2, Wc, bc):
    ei = edge_index.astype(jnp.int32)
    pad = jnp.full((E_PAD - E,), N, dtype=jnp.int32)
    src = jnp.concatenate([ei[0], pad]).reshape(E_PAD // CHUNK, CHUNK)
    dst = jnp.concatenate([ei[1], pad]).reshape(E_PAD // CHUNK, CHUNK)
    x_pad = jnp.pad(x, ((0, N_PAD - N), (0, 0)))

    t1 = _mm(x_pad, W1)
    deg = _deg_kernel(dst)
    hs1, dsp = _scale_t1_kernel(deg, t1)
    agg1 = _agg64(hs1, src, dst)
    z1 = _comb64(agg1, hs1, dsp, b1)
    t2 = _mm(z1, W2)
    hs2 = _scale_t2_kernel(dsp, t2)
    agg2 = _agg32(hs2, src, dst)
    z2 = _comb32(agg2, hs2, dsp, b2)
    out = _mm_bias(z2, Wc, bc)
    return out[:N]


# speedup vs baseline: 1.0070x; 1.0070x over previous
import functools

import jax
import jax.numpy as jnp
from jax import lax
from jax.experimental import pallas as pl
from jax.experimental.pallas import tpu as pltpu
from jax.experimental.pallas import tpu_sc as plsc

N = 10000
N_PAD = 10240
RPT = N_PAD // 32
NPT = N_PAD // 16
E = 320000
CHUNK = 128
CPT = 80
E_PAD = 32 * CHUNK * CPT
EPT = CHUNK * CPT
NBUF = 8
NGRP = CPT // NBUF

_MESH = plsc.VectorSubcoreMesh(core_axis_name="c", subcore_axis_name="s")
_SC_PARAMS = pltpu.CompilerParams(needs_layout_passes=False,
                                  use_tc_tiling_on_sc=False)


def _bit_rsqrt(v):
    i = plsc.bitcast(v, jnp.int32)
    i = jnp.int32(0x5F3759DF) - (i >> 1)
    y = plsc.bitcast(i, jnp.float32)
    for _ in range(3):
        y = y * (1.5 - 0.5 * v * y * y)
    return y


@functools.partial(
    pl.kernel,
    out_type=jax.ShapeDtypeStruct((2 * N_PAD, 16), jnp.float32),
    mesh=_MESH,
    compiler_params=_SC_PARAMS,
    scratch_types=[
        pltpu.VMEM((CPT, CHUNK), jnp.int32),
        pltpu.VMEM((CHUNK, 16), jnp.float32),
        pltpu.VMEM((CHUNK, 16), jnp.float32),
        pltpu.VMEM_SHARED((N_PAD, 16), jnp.float32),
    ] + [pltpu.SemaphoreType.DMA] * NBUF,
)
def _deg_kernel(dst_hbm, out_hbm, dst_all, ones_v, zeros_v, acc, *ssems):
    cid = lax.axis_index("c")
    sid = lax.axis_index("s")
    wid = cid * 16 + sid

    def fill(r, carry):
        ones_v[r, :] = jnp.full((16,), 1.0, jnp.float32)
        zeros_v[r, :] = jnp.zeros((16,), jnp.float32)
        return carry

    lax.fori_loop(0, CHUNK, fill, 0)
    pltpu.sync_copy(dst_hbm.at[pl.ds(wid * CPT, CPT), :], dst_all)
    for k in range(NPT // CHUNK):
        pltpu.sync_copy(zeros_v, acc.at[pl.ds(sid * NPT + k * CHUNK, CHUNK), :])
    plsc.subcore_barrier()

    def body(g, carry):
        descs = []
        for b in range(NBUF):
            descs.append(pltpu.async_copy(
                ones_v, acc.at[dst_all.at[g * NBUF + b]], ssems[b], add=True))
        for d in descs:
            d.wait()
        return carry

    lax.fori_loop(0, NGRP, body, 0)
    plsc.subcore_barrier()
    pltpu.sync_copy(acc.at[pl.ds(sid * NPT, NPT), :],
                    out_hbm.at[pl.ds(cid * N_PAD + sid * NPT, NPT), :])


@functools.partial(
    pl.kernel,
    out_type=(jax.ShapeDtypeStruct((N_PAD, 64), jnp.float32),
              jax.ShapeDtypeStruct((N_PAD, 16), jnp.float32)),
    mesh=_MESH,
    compiler_params=_SC_PARAMS,
    scratch_types=[
        pltpu.VMEM((RPT, 16), jnp.float32),
        pltpu.VMEM((RPT, 16), jnp.float32),
        pltpu.VMEM((RPT, 64), jnp.float32),
    ],
)
def _scale_t1_kernel(deg_hbm, t_hbm, hs_hbm, d_hbm, dg0, dg1, tbuf):
    cid = lax.axis_index("c")
    sid = lax.axis_index("s")
    r0 = (cid * 16 + sid) * RPT
    pltpu.sync_copy(deg_hbm.at[pl.ds(r0, RPT), :], dg0)
    pltpu.sync_copy(deg_hbm.at[pl.ds(N_PAD + r0, RPT), :], dg1)
    pltpu.sync_copy(t_hbm.at[pl.ds(r0, RPT), :], tbuf)

    def row(r, carry):
        deg = dg0[r, :] + dg1[r, :] + 1.0
        dv = _bit_rsqrt(deg)
        dg0[r, :] = dv
        for c in range(64 // 16):
            tbuf[r, pl.ds(c * 16, 16)] = tbuf[r, pl.ds(c * 16, 16)] * dv
        return carry

    lax.fori_loop(0, RPT, row, 0)
    pltpu.sync_copy(tbuf, hs_hbm.at[pl.ds(r0, RPT), :])
    pltpu.sync_copy(dg0, d_hbm.at[pl.ds(r0, RPT), :])


@functools.partial(
    pl.kernel,
    out_type=jax.ShapeDtypeStruct((N_PAD, 32), jnp.float32),
    mesh=_MESH,
    compiler_params=_SC_PARAMS,
    scratch_types=[
        pltpu.VMEM((RPT, 16), jnp.float32),
        pltpu.VMEM((RPT, 32), jnp.float32),
    ],
)
def _scale_t2_kernel(d_hbm, t_hbm, hs_hbm, db, tbuf):
    cid = lax.axis_index("c")
    sid = lax.axis_index("s")
    r0 = (cid * 16 + sid) * RPT
    pltpu.sync_copy(d_hbm.at[pl.ds(r0, RPT), :], db)
    pltpu.sync_copy(t_hbm.at[pl.ds(r0, RPT), :], tbuf)

    def row(r, carry):
        dv = db[r, :]
        for c in range(32 // 16):
            tbuf[r, pl.ds(c * 16, 16)] = tbuf[r, pl.ds(c * 16, 16)] * dv
        return carry

    lax.fori_loop(0, RPT, row, 0)
    pltpu.sync_copy(tbuf, hs_hbm.at[pl.ds(r0, RPT), :])


def _make_agg(D):
    nbuf = 2 if D == 64 else 8
    @functools.partial(
        pl.kernel,
        out_type=jax.ShapeDtypeStruct((2 * N_PAD, D), jnp.float32),
        mesh=_MESH,
        compiler_params=_SC_PARAMS,
        scratch_types=[
            pltpu.VMEM((CPT, CHUNK), jnp.int32),
            pltpu.VMEM((CPT, CHUNK), jnp.int32),
            pltpu.VMEM((nbuf, CHUNK, D), jnp.float32),
            pltpu.VMEM_SHARED((N_PAD, D), jnp.float32),
            pltpu.VMEM_SHARED((N_PAD, D), jnp.float32),
        ] + [pltpu.SemaphoreType.DMA] * (2 * nbuf),
    )
    def _agg(hs_hbm, src_hbm, dst_hbm, out_hbm, src_all, dst_all, rows,
             hs_cache, acc, *sems):
        gsems, ssems = sems[:nbuf], sems[nbuf:]
        cid = lax.axis_index("c")
        sid = lax.axis_index("s")
        wid = cid * 16 + sid

        def fill(r, carry):
            for c in range(D // 16):
                rows[0, r, pl.ds(c * 16, 16)] = jnp.zeros((16,), jnp.float32)
            return carry

        lax.fori_loop(0, CHUNK, fill, 0)
        pltpu.sync_copy(src_hbm.at[pl.ds(wid * CPT, CPT), :], src_all)
        pltpu.sync_copy(dst_hbm.at[pl.ds(wid * CPT, CPT), :], dst_all)
        pltpu.sync_copy(hs_hbm.at[pl.ds(sid * NPT, NPT), :],
                        hs_cache.at[pl.ds(sid * NPT, NPT), :])
        for k in range(NPT // CHUNK):
            pltpu.sync_copy(rows.at[0],
                            acc.at[pl.ds(sid * NPT + k * CHUNK, CHUNK), :])
        plsc.subcore_barrier()
        for b in range(nbuf):
            pltpu.async_copy(hs_cache.at[src_all.at[b]], rows.at[b], gsems[b])

        def body(g, carry):
            descs = []
            for b in range(nbuf):
                j = g * nbuf + b
                pltpu.make_async_copy(hs_cache.at[src_all.at[j]], rows.at[b],
                                      gsems[b]).wait()
                descs.append(pltpu.async_copy(
                    rows.at[b], acc.at[dst_all.at[j]], ssems[b], add=True))
            for d in descs:
                d.wait()

            @pl.when(g + 1 < CPT // nbuf)
            def _refill():
                for b in range(nbuf):
                    j2 = (g + 1) * nbuf + b
                    pltpu.async_copy(hs_cache.at[src_all.at[j2]], rows.at[b],
                                     gsems[b])

            return carry

        lax.fori_loop(0, CPT // nbuf, body, 0)
        plsc.subcore_barrier()
        pltpu.sync_copy(acc.at[pl.ds(sid * NPT, NPT), :],
                        out_hbm.at[pl.ds(cid * N_PAD + sid * NPT, NPT), :])

    return _agg


_agg64 = _make_agg(64)
_agg32 = _make_agg(32)


def _make_combine(D, extra_scale):
    @functools.partial(
        pl.kernel,
        out_type=jax.ShapeDtypeStruct((N_PAD, D), jnp.float32),
        mesh=_MESH,
        compiler_params=_SC_PARAMS,
        scratch_types=[
            pltpu.VMEM((RPT, D), jnp.float32),
            pltpu.VMEM((RPT, D), jnp.float32),
            pltpu.VMEM((RPT, D), jnp.float32),
            pltpu.VMEM((RPT, 16), jnp.float32),
            pltpu.VMEM((D,), jnp.float32),
        ],
    )
    def _comb(agg_hbm, hs_hbm, d_hbm, b_hbm, out_hbm, a0, a1, hsb, db, bb):
        cid = lax.axis_index("c")
        sid = lax.axis_index("s")
        r0 = (cid * 16 + sid) * RPT
        pltpu.sync_copy(agg_hbm.at[pl.ds(r0, RPT), :], a0)
        pltpu.sync_copy(agg_hbm.at[pl.ds(N_PAD + r0, RPT), :], a1)
        pltpu.sync_copy(hs_hbm.at[pl.ds(r0, RPT), :], hsb)
        pltpu.sync_copy(d_hbm.at[pl.ds(r0, RPT), :], db)
        pltpu.sync_copy(b_hbm, bb)

        def row(r, carry):
            dv = db[r, :]
            for c in range(D // 16):
                sl = pl.ds(c * 16, 16)
                s = (a0[r, sl] + a1[r, sl] + hsb[r, sl]) * dv + bb[sl]
                z = jnp.maximum(s, 0.0)
                if extra_scale:
                    z = z * dv
                a0[r, sl] = z
            return carry

        lax.fori_loop(0, RPT, row, 0)
        pltpu.sync_copy(a0, out_hbm.at[pl.ds(r0, RPT), :])

    return _comb


_comb64 = _make_combine(64, False)
_comb32 = _make_combine(32, False)


def _mm_body(x_ref, w_ref, o_ref):
    o_ref[...] = jnp.dot(x_ref[...], w_ref[...],
                         preferred_element_type=jnp.float32)


def _mm(xv, wv):
    return pl.pallas_call(
        _mm_body,
        out_shape=jax.ShapeDtypeStruct((xv.shape[0], wv.shape[1]),
                                       jnp.float32),
    )(xv, wv)


def _mm_bias_body(x_ref, w_ref, b_ref, o_ref):
    o_ref[...] = jnp.dot(x_ref[...], w_ref[...],
                         preferred_element_type=jnp.float32) + b_ref[...]


def _mm_bias(xv, wv, bv):
    return pl.pallas_call(
        _mm_bias_body,
        out_shape=jax.ShapeDtypeStruct((xv.shape[0], wv.shape[1]),
                                       jnp.float32),
    )(xv, wv, bv)


def kernel(x, edge_index, W1, b1, W2, b2, Wc, bc):
    ei = edge_index.astype(jnp.int32)
    pad = jnp.full((E_PAD - E,), N, dtype=jnp.int32)
    src = jnp.concatenate([ei[0], pad]).reshape(E_PAD // CHUNK, CHUNK)
    dst = jnp.concatenate([ei[1], pad]).reshape(E_PAD // CHUNK, CHUNK)
    x_pad = jnp.pad(x, ((0, N_PAD - N), (0, 0)))

    t1 = _mm(x_pad, W1)
    deg = _deg_kernel(dst)
    hs1, dsp = _scale_t1_kernel(deg, t1)
    agg1 = _agg64(hs1, src, dst)
    z1 = _comb64(agg1, hs1, dsp, b1)
    t2 = _mm(z1, W2)
    hs2 = _scale_t2_kernel(dsp, t2)
    agg2 = _agg32(hs2, src, dst)
    z2 = _comb32(agg2, hs2, dsp, b2)
    out = _mm_bias(z2, Wc, bc)
    return out[:N]
